# baseline (speedup 1.0000x reference)
import jax
import jax.numpy as jnp
import numpy as np
from jax import lax
from jax.experimental import pallas as pl
from jax.experimental.pallas import tpu as pltpu
from jax.experimental.pallas import tpu_sc as plsc

T = 32768
E = 256
NW = 32
TPW = T // NW
C = 64
NCHUNK = TPW // C
NBLK = C // 16
BLKS = E * 17
NEG = float("-inf")
TOP_K = 8
SCALE = 2.5


def _router_body(logits_hbm, bias_hbm, outv_hbm, outi_hbm,
                 bias_v, s_T, swb_T, selid_T, chunk, valbuf, idxbuf):
    wid = lax.axis_index("s") * 2 + lax.axis_index("c")
    base = wid * TPW

    pltpu.sync_copy(bias_hbm, bias_v)
    lane = lax.iota(jnp.int32, 16)
    zeros_i = jnp.zeros((16,), jnp.int32)
    negvec = jnp.full((16,), NEG, jnp.float32)
    lane8 = lane < 8
    lane17 = lane * 17
    gidx = [jnp.full((16,), g, jnp.int32) for g in range(8)]

    def block_body(b, carry):
        boffs = b * BLKS

        def e_body(e, c):
            ms = list(c)
            for g in range(8):
                eg = g * 32 + e
                v = swb_T[pl.ds(boffs + eg * 17, 16)]
                m1, m2 = ms[2 * g], ms[2 * g + 1]
                ms[2 * g] = jnp.maximum(m1, v)
                ms[2 * g + 1] = jnp.maximum(m2, jnp.minimum(m1, v))
            return tuple(ms)

        ms = lax.fori_loop(0, 32, e_body, (negvec,) * 16)
        gs = [ms[2 * g] + ms[2 * g + 1] for g in range(8)]
        for r in range(4):
            vals = list(gs)
            idxs = list(gidx)
            while len(vals) > 1:
                nv, ni = [], []
                for i in range(len(vals) // 2):
                    a, bb = vals[2 * i], vals[2 * i + 1]
                    ia, ib = idxs[2 * i], idxs[2 * i + 1]
                    take = a >= bb
                    nv.append(jnp.where(take, a, bb))
                    ni.append(jnp.where(take, ia, ib))
                vals, idxs = nv, ni
            win = idxs[0]
            selid_T[pl.ds(b * 64 + r * 16, 16)] = win
            gs = [jnp.where(win == g, negvec, gs[g]) for g in range(8)]
        return carry

    def tok_body(t):
        b = t >> 4
        tr = t & 15
        soff = b * BLKS + tr
        lists = []
        for r in range(4):
            gid = plsc.load_gather(selid_T, [zeros_i + (b * 64 + r * 16 + tr)])
            for h in range(2):
                ei = gid * 32 + h * 16 + lane
                swb = plsc.load_gather(swb_T, [ei * 17 + soff])
                lists.append(plsc.sort_key_val(swb, ei, descending=True))
        while len(lists) > 1:
            nxt = []
            for i in range(len(lists) // 2):
                ak, ai = lists[2 * i]
                bk, bi = lists[2 * i + 1]
                brk = lax.rev(bk, (0,))
                bri = lax.rev(bi, (0,))
                take = ak >= brk
                hk = jnp.where(take, ak, brk)
                hi = jnp.where(take, ai, bri)
                nxt.append(plsc.sort_key_val(hk, hi, descending=True))
            lists = nxt
        tk, ti = lists[0]
        sg = plsc.load_gather(s_T, [ti * 17 + soff])
        gsel = jnp.where(lane8, sg, 0.0)
        ssum = jnp.sum(gsel) + 1e-20
        ov = gsel / ssum * SCALE
        fk, fi = plsc.sort_key_val(ov, ti, descending=True)
        plsc.store_compressed(valbuf.at[pl.ds(t * 8, 16)], fk, mask=lane8)
        plsc.store_compressed(idxbuf.at[pl.ds(t * 8, 16)], fi, mask=lane8)

    def sig_body(t):
        rowoff = t * E
        dst0 = (t >> 4) * BLKS + (t & 15)
        for j in range(16):
            x = chunk[pl.ds(rowoff + 16 * j, 16)]
            s = 1.0 / (1.0 + jnp.exp(-x))
            idx = lane17 + (dst0 + 272 * j)
            plsc.store_scatter(s_T, [idx], s)
            plsc.store_scatter(swb_T, [idx], s + bias_v[pl.ds(16 * j, 16)])

    def chunk_body(ci, carry):
        row0 = (base + ci * C) * E
        pltpu.sync_copy(logits_hbm.at[pl.ds(row0, C * E)], chunk)
        plsc.parallel_loop(0, C, 1, unroll=2)(sig_body)
        lax.fori_loop(0, NBLK, block_body, 0)
        plsc.parallel_loop(0, C, 1, unroll=2)(tok_body)
        o = (base + ci * C) * 8
        pltpu.sync_copy(valbuf.at[pl.ds(0, C * 8)],
                        outv_hbm.at[pl.ds(o, C * 8)])
        pltpu.sync_copy(idxbuf.at[pl.ds(0, C * 8)],
                        outi_hbm.at[pl.ds(o, C * 8)])
        return carry

    lax.fori_loop(0, NCHUNK, chunk_body, 0)


@jax.jit
def kernel(logits, e_score_correction_bias):
    mesh = plsc.VectorSubcoreMesh(core_axis_name="c", subcore_axis_name="s",
                                  num_cores=2, num_subcores=16)
    f = pl.kernel(
        _router_body,
        out_type=[
            jax.ShapeDtypeStruct((T * 8,), jnp.float32),
            jax.ShapeDtypeStruct((T * 8,), jnp.int32),
        ],
        mesh=mesh,
        compiler_params=pltpu.CompilerParams(needs_layout_passes=False),
        scratch_types=[
            pltpu.VMEM((E,), jnp.float32),
            pltpu.VMEM((NBLK * BLKS,), jnp.float32),
            pltpu.VMEM((NBLK * BLKS,), jnp.float32),
            pltpu.VMEM((NBLK * 64,), jnp.int32),
            pltpu.VMEM((C * E,), jnp.float32),
            pltpu.VMEM((C * 8 + 8,), jnp.float32),
            pltpu.VMEM((C * 8 + 8,), jnp.int32),
        ],
    )
    vals, idxs = f(logits.reshape(-1), e_score_correction_bias)
    return vals.reshape(T, TOP_K), idxs.reshape(T, TOP_K)


# speedup vs baseline: 1.0374x; 1.0374x over previous
import jax
import jax.numpy as jnp
import numpy as np
from jax import lax
from jax.experimental import pallas as pl
from jax.experimental.pallas import tpu as pltpu
from jax.experimental.pallas import tpu_sc as plsc

T = 32768
E = 256
NW = 32
TPW = T // NW
C = 64
NCHUNK = TPW // C
NBLK = C // 16
BLKS = E * 17
NEG = float("-inf")
TOP_K = 8
SCALE = 2.5


def _router_body(logits_hbm, bias_hbm, outv_hbm, outi_hbm,
                 bias_v, s_T, swb_T, selid_T, chunk, valbuf, idxbuf):
    wid = lax.axis_index("s") * 2 + lax.axis_index("c")
    base = wid * TPW

    pltpu.sync_copy(bias_hbm, bias_v)
    lane = lax.iota(jnp.int32, 16)
    zeros_i = jnp.zeros((16,), jnp.int32)
    negvec = jnp.full((16,), NEG, jnp.float32)
    lane8 = lane < 8
    lane17 = lane * 17
    gidx = [jnp.full((16,), g, jnp.int32) for g in range(8)]

    def block_body(b, carry):
        boffs = b * BLKS

        def e_body(e, c):
            ms = list(c)
            for g in range(8):
                eg = g * 32 + e
                v = swb_T[pl.ds(boffs + eg * 17, 16)]
                m1, m2 = ms[2 * g], ms[2 * g + 1]
                ms[2 * g] = jnp.maximum(m1, v)
                ms[2 * g + 1] = jnp.maximum(m2, jnp.minimum(m1, v))
            return tuple(ms)

        ms = lax.fori_loop(0, 32, e_body, (negvec,) * 16)
        gs = [ms[2 * g] + ms[2 * g + 1] for g in range(8)]
        for r in range(4):
            vals = list(gs)
            idxs = list(gidx)
            while len(vals) > 1:
                nv, ni = [], []
                for i in range(len(vals) // 2):
                    a, bb = vals[2 * i], vals[2 * i + 1]
                    ia, ib = idxs[2 * i], idxs[2 * i + 1]
                    take = a >= bb
                    nv.append(jnp.where(take, a, bb))
                    ni.append(jnp.where(take, ia, ib))
                vals, idxs = nv, ni
            win = idxs[0]
            selid_T[pl.ds(b * 64 + r * 16, 16)] = win
            gs = [jnp.where(win == g, negvec, gs[g]) for g in range(8)]
        return carry

    def tok_body(t):
        b = t >> 4
        tr = t & 15
        soff = b * BLKS + tr
        lists = []
        for r in range(4):
            gid = plsc.load_gather(selid_T, [zeros_i + (b * 64 + r * 16 + tr)])
            for h in range(2):
                ei = gid * 32 + h * 16 + lane
                swb = plsc.load_gather(swb_T, [ei * 17 + soff])
                lists.append(plsc.sort_key_val(swb, ei, descending=True))
        while len(lists) > 1:
            nxt = []
            for i in range(len(lists) // 2):
                ak, ai = lists[2 * i]
                bk, bi = lists[2 * i + 1]
                brk = lax.rev(bk, (0,))
                bri = lax.rev(bi, (0,))
                take = ak >= brk
                hk = jnp.where(take, ak, brk)
                hi = jnp.where(take, ai, bri)
                nxt.append(plsc.sort_key_val(hk, hi, descending=True))
            lists = nxt
        tk, ti = lists[0]
        sg = plsc.load_gather(s_T, [ti * 17 + soff])
        gsel = jnp.where(lane8, sg, 0.0)
        ssum = jnp.sum(gsel) + 1e-20
        ov = gsel / ssum * SCALE
        fk, fi = plsc.sort_key_val(ov, ti, descending=True)
        plsc.store_compressed(valbuf.at[pl.ds(t * 8, 16)], fk, mask=lane8)
        plsc.store_compressed(idxbuf.at[pl.ds(t * 8, 16)], fi, mask=lane8)

    def sig_body(t):
        rowoff = (t >> 3) * 2048 + (t & 7) * 128
        dst0 = (t >> 4) * BLKS + (t & 15)
        for j in range(16):
            x = chunk[pl.ds(rowoff + (j >> 3) * 1024 + (j & 7) * 16, 16)]
            s = 1.0 / (1.0 + jnp.exp(-x))
            idx = lane17 + (dst0 + 272 * j)
            plsc.store_scatter(s_T, [idx], s)
            plsc.store_scatter(swb_T, [idx], s + bias_v[pl.ds(16 * j, 16)])

    def chunk_body(ci, carry):
        row0 = (base + ci * C) * E
        pltpu.sync_copy(logits_hbm.at[pl.ds(row0, C * E)], chunk)
        plsc.parallel_loop(0, C, 1, unroll=2)(sig_body)
        lax.fori_loop(0, NBLK, block_body, 0)
        plsc.parallel_loop(0, C, 1, unroll=2)(tok_body)
        o = (base + ci * C) * 8
        pltpu.sync_copy(valbuf.at[pl.ds(0, C * 8)],
                        outv_hbm.at[pl.ds(o, C * 8)])
        pltpu.sync_copy(idxbuf.at[pl.ds(0, C * 8)],
                        outi_hbm.at[pl.ds(o, C * 8)])
        return carry

    lax.fori_loop(0, NCHUNK, chunk_body, 0)


@jax.jit
def kernel(logits, e_score_correction_bias):
    mesh = plsc.VectorSubcoreMesh(core_axis_name="c", subcore_axis_name="s",
                                  num_cores=2, num_subcores=16)
    f = pl.kernel(
        _router_body,
        out_type=[
            jax.ShapeDtypeStruct((T * 8,), jnp.float32),
            jax.ShapeDtypeStruct((T * 8,), jnp.int32),
        ],
        mesh=mesh,
        compiler_params=pltpu.CompilerParams(needs_layout_passes=False),
        scratch_types=[
            pltpu.VMEM((E,), jnp.float32),
            pltpu.VMEM((NBLK * BLKS,), jnp.float32),
            pltpu.VMEM((NBLK * BLKS,), jnp.float32),
            pltpu.VMEM((NBLK * 64,), jnp.int32),
            pltpu.VMEM((C * E,), jnp.float32),
            pltpu.VMEM((C * 8 + 8,), jnp.float32),
            pltpu.VMEM((C * 8 + 8,), jnp.int32),
        ],
    )
    xt = logits.reshape(4096, 8, 2, 128).transpose(0, 2, 1, 3).reshape(-1)
    vals, idxs = f(xt, e_score_correction_bias)
    return vals.reshape(T, TOP_K), idxs.reshape(T, TOP_K)


# speedup vs baseline: 1.0699x; 1.0313x over previous
import jax
import jax.numpy as jnp
import numpy as np
from jax import lax
from jax.experimental import pallas as pl
from jax.experimental.pallas import tpu as pltpu
from jax.experimental.pallas import tpu_sc as plsc

T = 32768
E = 256
NW = 32
TPW = T // NW
C = 64
NCHUNK = TPW // C
NBLK = C // 16
BLKS = E * 17
NEG = float("-inf")
TOP_K = 8
SCALE = 2.5


def _router_body(logits_hbm, bias_hbm, outv_hbm, outi_hbm,
                 bias_v, s_T, swb_T, selid_T, chunkA, chunkB, valbuf, idxbuf,
                 semA, semB):
    wid = lax.axis_index("s") * 2 + lax.axis_index("c")
    base = wid * TPW

    pltpu.sync_copy(bias_hbm, bias_v)
    lane = lax.iota(jnp.int32, 16)
    zeros_i = jnp.zeros((16,), jnp.int32)
    negvec = jnp.full((16,), NEG, jnp.float32)
    lane8 = lane < 8
    lane17 = lane * 17
    gidx = [jnp.full((16,), g, jnp.int32) for g in range(8)]

    def block_body(b, carry):
        boffs = b * BLKS

        def e_body(e, c):
            ms = list(c)
            for g in range(8):
                eg = g * 32 + e
                v = swb_T[pl.ds(boffs + eg * 17, 16)]
                m1, m2 = ms[2 * g], ms[2 * g + 1]
                ms[2 * g] = jnp.maximum(m1, v)
                ms[2 * g + 1] = jnp.maximum(m2, jnp.minimum(m1, v))
            return tuple(ms)

        ms = lax.fori_loop(0, 32, e_body, (negvec,) * 16)
        gs = [ms[2 * g] + ms[2 * g + 1] for g in range(8)]
        for r in range(4):
            vals = list(gs)
            idxs = list(gidx)
            while len(vals) > 1:
                nv, ni = [], []
                for i in range(len(vals) // 2):
                    a, bb = vals[2 * i], vals[2 * i + 1]
                    ia, ib = idxs[2 * i], idxs[2 * i + 1]
                    take = a >= bb
                    nv.append(jnp.where(take, a, bb))
                    ni.append(jnp.where(take, ia, ib))
                vals, idxs = nv, ni
            win = idxs[0]
            selid_T[pl.ds(b * 64 + r * 16, 16)] = win
            gs = [jnp.where(win == g, negvec, gs[g]) for g in range(8)]
        return carry

    def tok_body(t):
        b = t >> 4
        tr = t & 15
        soff = b * BLKS + tr
        lists = []
        for r in range(4):
            gid = plsc.load_gather(selid_T, [zeros_i + (b * 64 + r * 16 + tr)])
            for h in range(2):
                ei = gid * 32 + h * 16 + lane
                swb = plsc.load_gather(swb_T, [ei * 17 + soff])
                lists.append(plsc.sort_key_val(swb, ei, descending=True))
        while len(lists) > 1:
            nxt = []
            for i in range(len(lists) // 2):
                ak, ai = lists[2 * i]
                bk, bi = lists[2 * i + 1]
                brk = lax.rev(bk, (0,))
                bri = lax.rev(bi, (0,))
                take = ak >= brk
                hk = jnp.where(take, ak, brk)
                hi = jnp.where(take, ai, bri)
                nxt.append(plsc.sort_key_val(hk, hi, descending=True))
            lists = nxt
        tk, ti = lists[0]
        sg = plsc.load_gather(s_T, [ti * 17 + soff])
        gsel = jnp.where(lane8, sg, 0.0)
        ssum = jnp.sum(gsel) + 1e-20
        ov = gsel / ssum * SCALE
        fk, fi = plsc.sort_key_val(ov, ti, descending=True)
        plsc.store_compressed(valbuf.at[pl.ds(t * 8, 16)], fk, mask=lane8)
        plsc.store_compressed(idxbuf.at[pl.ds(t * 8, 16)], fi, mask=lane8)

    def make_sig_body(chunk):
        def sig_body(t):
            rowoff = (t >> 3) * 2048 + (t & 7) * 128
            dst0 = (t >> 4) * BLKS + (t & 15)
            for j in range(16):
                x = chunk[pl.ds(rowoff + (j >> 3) * 1024 + (j & 7) * 16, 16)]
                s = 1.0 / (1.0 + jnp.exp(-x))
                idx = lane17 + (dst0 + 272 * j)
                plsc.store_scatter(s_T, [idx], s)
                plsc.store_scatter(swb_T, [idx], s + bias_v[pl.ds(16 * j, 16)])
        return sig_body

    def in_copy(ci, buf, sem):
        row0 = (base + ci * C) * E
        return pltpu.make_async_copy(logits_hbm.at[pl.ds(row0, C * E)],
                                     buf, sem)

    def chunk_compute(ci, chunk):
        plsc.parallel_loop(0, C, 1, unroll=2)(make_sig_body(chunk))
        lax.fori_loop(0, NBLK, block_body, 0)
        plsc.parallel_loop(0, C, 1, unroll=2)(tok_body)
        o = (base + ci * C) * 8
        pltpu.sync_copy(valbuf.at[pl.ds(0, C * 8)],
                        outv_hbm.at[pl.ds(o, C * 8)])
        pltpu.sync_copy(idxbuf.at[pl.ds(0, C * 8)],
                        outi_hbm.at[pl.ds(o, C * 8)])

    in_copy(0, chunkA, semA).start()

    def pair_body(cp, carry):
        ci0 = cp * 2
        in_copy(ci0 + 1, chunkB, semB).start()
        in_copy(ci0, chunkA, semA).wait()
        chunk_compute(ci0, chunkA)

        @pl.when(cp < NCHUNK // 2 - 1)
        def _():
            in_copy(ci0 + 2, chunkA, semA).start()

        in_copy(ci0 + 1, chunkB, semB).wait()
        chunk_compute(ci0 + 1, chunkB)
        return carry

    lax.fori_loop(0, NCHUNK // 2, pair_body, 0)


@jax.jit
def kernel(logits, e_score_correction_bias):
    mesh = plsc.VectorSubcoreMesh(core_axis_name="c", subcore_axis_name="s",
                                  num_cores=2, num_subcores=16)
    f = pl.kernel(
        _router_body,
        out_type=[
            jax.ShapeDtypeStruct((T * 8,), jnp.float32),
            jax.ShapeDtypeStruct((T * 8,), jnp.int32),
        ],
        mesh=mesh,
        compiler_params=pltpu.CompilerParams(needs_layout_passes=False),
        scratch_types=[
            pltpu.VMEM((E,), jnp.float32),
            pltpu.VMEM((NBLK * BLKS,), jnp.float32),
            pltpu.VMEM((NBLK * BLKS,), jnp.float32),
            pltpu.VMEM((NBLK * 64,), jnp.int32),
            pltpu.VMEM((C * E,), jnp.float32),
            pltpu.VMEM((C * E,), jnp.float32),
            pltpu.VMEM((C * 8 + 8,), jnp.float32),
            pltpu.VMEM((C * 8 + 8,), jnp.int32),
            pltpu.SemaphoreType.DMA,
            pltpu.SemaphoreType.DMA,
        ],
    )
    xt = logits.reshape(4096, 8, 2, 128).transpose(0, 2, 1, 3).reshape(-1)
    vals, idxs = f(xt, e_score_correction_bias)
    return vals.reshape(T, TOP_K), idxs.reshape(T, TOP_K)


# speedup vs baseline: 1.0767x; 1.0063x over previous
import jax
import jax.numpy as jnp
import numpy as np
from jax import lax
from jax.experimental import pallas as pl
from jax.experimental.pallas import tpu as pltpu
from jax.experimental.pallas import tpu_sc as plsc

T = 32768
E = 256
NW = 32
TPW = T // NW
C = 64
NCHUNK = TPW // C
NBLK = C // 16
BLKS = E * 17
NEG = float("-inf")
TOP_K = 8
SCALE = 2.5


def _router_body(logits_hbm, bias_hbm, outv_hbm, outi_hbm,
                 bias_v, s_T, swb_T, selid_T, chunkA, chunkB, valbuf, idxbuf,
                 semA, semB, semO):
    wid = lax.axis_index("s") * 2 + lax.axis_index("c")
    base = wid * TPW

    pltpu.sync_copy(bias_hbm, bias_v)
    lane = lax.iota(jnp.int32, 16)
    zeros_i = jnp.zeros((16,), jnp.int32)
    negvec = jnp.full((16,), NEG, jnp.float32)
    lane8 = lane < 8
    lane17 = lane * 17
    gidx = [jnp.full((16,), g, jnp.int32) for g in range(8)]

    def block_body(b, carry):
        boffs = b * BLKS

        def e_body(e, c):
            ms = list(c)
            for g in range(8):
                eg = g * 32 + e
                v = swb_T[pl.ds(boffs + eg * 17, 16)]
                m1, m2 = ms[2 * g], ms[2 * g + 1]
                ms[2 * g] = jnp.maximum(m1, v)
                ms[2 * g + 1] = jnp.maximum(m2, jnp.minimum(m1, v))
            return tuple(ms)

        ms = lax.fori_loop(0, 32, e_body, (negvec,) * 16)
        gs = [ms[2 * g] + ms[2 * g + 1] for g in range(8)]
        for r in range(4):
            vals = list(gs)
            idxs = list(gidx)
            while len(vals) > 1:
                nv, ni = [], []
                for i in range(len(vals) // 2):
                    a, bb = vals[2 * i], vals[2 * i + 1]
                    ia, ib = idxs[2 * i], idxs[2 * i + 1]
                    take = a >= bb
                    nv.append(jnp.where(take, a, bb))
                    ni.append(jnp.where(take, ia, ib))
                vals, idxs = nv, ni
            win = idxs[0]
            selid_T[pl.ds(b * 64 + r * 16, 16)] = win
            gs = [jnp.where(win == g, negvec, gs[g]) for g in range(8)]
        return carry

    def tok_body(t):
        b = t >> 4
        tr = t & 15
        soff = b * BLKS + tr
        lists = []
        for r in range(4):
            gid = plsc.load_gather(selid_T, [zeros_i + (b * 64 + r * 16 + tr)])
            for h in range(2):
                ei = gid * 32 + h * 16 + lane
                swb = plsc.load_gather(swb_T, [ei * 17 + soff])
                lists.append(plsc.sort_key_val(swb, ei, descending=True))
        while len(lists) > 1:
            nxt = []
            for i in range(len(lists) // 2):
                ak, ai = lists[2 * i]
                bk, bi = lists[2 * i + 1]
                brk = lax.rev(bk, (0,))
                bri = lax.rev(bi, (0,))
                take = ak >= brk
                hk = jnp.where(take, ak, brk)
                hi = jnp.where(take, ai, bri)
                nxt.append(plsc.sort_key_val(hk, hi, descending=True))
            lists = nxt
        tk, ti = lists[0]
        sg = plsc.load_gather(s_T, [ti * 17 + soff])
        gsel = jnp.where(lane8, sg, 0.0)
        ssum = jnp.sum(gsel) + 1e-20
        ov = gsel / ssum * SCALE
        fk, fi = plsc.sort_key_val(ov, ti, descending=True)
        plsc.store_compressed(valbuf.at[pl.ds(t * 8, 16)], fk, mask=lane8)
        plsc.store_compressed(idxbuf.at[pl.ds(t * 8, 16)], fi, mask=lane8)

    def make_sig_body(chunk):
        def sig_body(t):
            rowoff = (t >> 3) * 2048 + (t & 7) * 128
            dst0 = (t >> 4) * BLKS + (t & 15)
            for j in range(16):
                x = chunk[pl.ds(rowoff + (j >> 3) * 1024 + (j & 7) * 16, 16)]
                s = 1.0 / (1.0 + jnp.exp(-x))
                idx = lane17 + (dst0 + 272 * j)
                plsc.store_scatter(s_T, [idx], s)
                plsc.store_scatter(swb_T, [idx], s + bias_v[pl.ds(16 * j, 16)])
        return sig_body

    def in_copy(ci, buf, sem):
        row0 = (base + ci * C) * E
        return pltpu.make_async_copy(logits_hbm.at[pl.ds(row0, C * E)],
                                     buf, sem)

    def out_copies(ci):
        o = (base + ci * C) * 8
        return (pltpu.make_async_copy(valbuf.at[pl.ds(0, C * 8)],
                                      outv_hbm.at[pl.ds(o, C * 8)], semO),
                pltpu.make_async_copy(idxbuf.at[pl.ds(0, C * 8)],
                                      outi_hbm.at[pl.ds(o, C * 8)], semO))

    def chunk_compute(ci, chunk):
        plsc.parallel_loop(0, C, 1, unroll=2)(make_sig_body(chunk))
        lax.fori_loop(0, NBLK, block_body, 0)

        @pl.when(ci > 0)
        def _():
            cv, cx = out_copies(ci - 1)
            cv.wait()
            cx.wait()

        plsc.parallel_loop(0, C, 1, unroll=2)(tok_body)
        cv, cx = out_copies(ci)
        cv.start()
        cx.start()

    in_copy(0, chunkA, semA).start()

    def pair_body(cp, carry):
        ci0 = cp * 2
        in_copy(ci0 + 1, chunkB, semB).start()
        in_copy(ci0, chunkA, semA).wait()
        chunk_compute(ci0, chunkA)

        @pl.when(cp < NCHUNK // 2 - 1)
        def _():
            in_copy(ci0 + 2, chunkA, semA).start()

        in_copy(ci0 + 1, chunkB, semB).wait()
        chunk_compute(ci0 + 1, chunkB)
        return carry

    lax.fori_loop(0, NCHUNK // 2, pair_body, 0)
    cv, cx = out_copies(NCHUNK - 1)
    cv.wait()
    cx.wait()


@jax.jit
def kernel(logits, e_score_correction_bias):
    mesh = plsc.VectorSubcoreMesh(core_axis_name="c", subcore_axis_name="s",
                                  num_cores=2, num_subcores=16)
    f = pl.kernel(
        _router_body,
        out_type=[
            jax.ShapeDtypeStruct((T * 8,), jnp.float32),
            jax.ShapeDtypeStruct((T * 8,), jnp.int32),
        ],
        mesh=mesh,
        compiler_params=pltpu.CompilerParams(needs_layout_passes=False),
        scratch_types=[
            pltpu.VMEM((E,), jnp.float32),
            pltpu.VMEM((NBLK * BLKS,), jnp.float32),
            pltpu.VMEM((NBLK * BLKS,), jnp.float32),
            pltpu.VMEM((NBLK * 64,), jnp.int32),
            pltpu.VMEM((C * E,), jnp.float32),
            pltpu.VMEM((C * E,), jnp.float32),
            pltpu.VMEM((C * 8 + 8,), jnp.float32),
            pltpu.VMEM((C * 8 + 8,), jnp.int32),
            pltpu.SemaphoreType.DMA,
            pltpu.SemaphoreType.DMA,
            pltpu.SemaphoreType.DMA,
        ],
    )
    xt = logits.reshape(4096, 8, 2, 128).transpose(0, 2, 1, 3).reshape(-1)
    vals, idxs = f(xt, e_score_correction_bias)
    return vals.reshape(T, TOP_K), idxs.reshape(T, TOP_K)


# speedup vs baseline: 1.0825x; 1.0054x over previous
import jax
import jax.numpy as jnp
import numpy as np
from jax import lax
from jax.experimental import pallas as pl
from jax.experimental.pallas import tpu as pltpu
from jax.experimental.pallas import tpu_sc as plsc

T = 32768
E = 256
NW = 32
TPW = T // NW
C = 64
NCHUNK = TPW // C
NBLK = C // 16
BLKS = E * 17
NEG = float("-inf")
TOP_K = 8
SCALE = 2.5


def _router_body(logits_hbm, bias_hbm, outv_hbm, outi_hbm,
                 bias_v, s_T, swb_T, selid_T, chunkA, chunkB, valbuf, idxbuf,
                 semA, semB, semO):
    wid = lax.axis_index("s") * 2 + lax.axis_index("c")
    base = wid * TPW

    pltpu.sync_copy(bias_hbm, bias_v)
    lane = lax.iota(jnp.int32, 16)
    zeros_i = jnp.zeros((16,), jnp.int32)
    negvec = jnp.full((16,), NEG, jnp.float32)
    lane8 = lane < 8
    lane17 = lane * 17
    gidx = [jnp.full((16,), g, jnp.int32) for g in range(8)]

    def block_body(b, carry):
        boffs = b * BLKS

        def e_body(e, c):
            ms = list(c)
            for g in range(8):
                eg = g * 32 + e
                v = swb_T[pl.ds(boffs + eg * 17, 16)]
                m1, m2 = ms[2 * g], ms[2 * g + 1]
                ms[2 * g] = jnp.maximum(m1, v)
                ms[2 * g + 1] = jnp.maximum(m2, jnp.minimum(m1, v))
            return tuple(ms)

        ms = lax.fori_loop(0, 32, e_body, (negvec,) * 16)
        gs = [ms[2 * g] + ms[2 * g + 1] for g in range(8)]
        for r in range(4):
            vals = list(gs)
            idxs = list(gidx)
            while len(vals) > 1:
                nv, ni = [], []
                for i in range(len(vals) // 2):
                    a, bb = vals[2 * i], vals[2 * i + 1]
                    ia, ib = idxs[2 * i], idxs[2 * i + 1]
                    take = a >= bb
                    nv.append(jnp.where(take, a, bb))
                    ni.append(jnp.where(take, ia, ib))
                vals, idxs = nv, ni
            win = idxs[0]
            selid_T[pl.ds(b * 64 + r * 16, 16)] = win
            gs = [jnp.where(win == g, negvec, gs[g]) for g in range(8)]
        return carry

    def tok_body(t):
        b = t >> 4
        tr = t & 15
        soff = b * BLKS + tr
        lists = []
        for r in range(4):
            gid = plsc.load_gather(selid_T, [zeros_i + (b * 64 + r * 16 + tr)])
            for h in range(2):
                ei = gid * 32 + h * 16 + lane
                swb = plsc.load_gather(swb_T, [ei * 17 + soff])
                lists.append(plsc.sort_key_val(swb, ei, descending=True))
        while len(lists) > 1:
            nxt = []
            for i in range(len(lists) // 2):
                ak, ai = lists[2 * i]
                bk, bi = lists[2 * i + 1]
                brk = lax.rev(bk, (0,))
                bri = lax.rev(bi, (0,))
                take = ak >= brk
                hk = jnp.where(take, ak, brk)
                hi = jnp.where(take, ai, bri)
                nxt.append(plsc.sort_key_val(hk, hi, descending=True))
            lists = nxt
        tk, ti = lists[0]
        sg = plsc.load_gather(s_T, [ti * 17 + soff])
        gsel = jnp.where(lane8, sg, 0.0)
        ssum = jnp.sum(gsel) + 1e-20
        ov = gsel / ssum * SCALE
        fk, fi = plsc.sort_key_val(ov, ti, descending=True)
        plsc.store_compressed(valbuf.at[pl.ds(t * 8, 16)], fk, mask=lane8)
        plsc.store_compressed(idxbuf.at[pl.ds(t * 8, 16)], fi, mask=lane8)

    def make_sig_body(chunk):
        def sig_body(t):
            dst0 = (t >> 4) * BLKS + (t & 15)
            a = t >> 3
            r = t & 7
            for j in range(16):
                x = chunk[a, j >> 3, r, pl.ds((j & 7) * 16, 16)]
                s = 1.0 / (1.0 + jnp.exp(-x))
                idx = lane17 + (dst0 + 272 * j)
                plsc.store_scatter(s_T, [idx], s)
                plsc.store_scatter(swb_T, [idx], s + bias_v[pl.ds(16 * j, 16)])
        return sig_body

    def in_copy(ci, buf, sem):
        a0 = (base + ci * C) >> 3
        return pltpu.make_async_copy(logits_hbm.at[pl.ds(a0, C // 8)],
                                     buf, sem)

    def out_copies(ci):
        o = (base + ci * C) * 8
        return (pltpu.make_async_copy(valbuf.at[pl.ds(0, C * 8)],
                                      outv_hbm.at[pl.ds(o, C * 8)], semO),
                pltpu.make_async_copy(idxbuf.at[pl.ds(0, C * 8)],
                                      outi_hbm.at[pl.ds(o, C * 8)], semO))

    def chunk_compute(ci, chunk):
        plsc.parallel_loop(0, C, 1, unroll=2)(make_sig_body(chunk))
        lax.fori_loop(0, NBLK, block_body, 0)

        @pl.when(ci > 0)
        def _():
            cv, cx = out_copies(ci - 1)
            cv.wait()
            cx.wait()

        plsc.parallel_loop(0, C, 1, unroll=2)(tok_body)
        cv, cx = out_copies(ci)
        cv.start()
        cx.start()

    in_copy(0, chunkA, semA).start()

    def pair_body(cp, carry):
        ci0 = cp * 2
        in_copy(ci0 + 1, chunkB, semB).start()
        in_copy(ci0, chunkA, semA).wait()
        chunk_compute(ci0, chunkA)

        @pl.when(cp < NCHUNK // 2 - 1)
        def _():
            in_copy(ci0 + 2, chunkA, semA).start()

        in_copy(ci0 + 1, chunkB, semB).wait()
        chunk_compute(ci0 + 1, chunkB)
        return carry

    lax.fori_loop(0, NCHUNK // 2, pair_body, 0)
    cv, cx = out_copies(NCHUNK - 1)
    cv.wait()
    cx.wait()


@jax.jit
def kernel(logits, e_score_correction_bias):
    mesh = plsc.VectorSubcoreMesh(core_axis_name="c", subcore_axis_name="s",
                                  num_cores=2, num_subcores=16)
    f = pl.kernel(
        _router_body,
        out_type=[
            jax.ShapeDtypeStruct((T * 8,), jnp.float32),
            jax.ShapeDtypeStruct((T * 8,), jnp.int32),
        ],
        mesh=mesh,
        compiler_params=pltpu.CompilerParams(needs_layout_passes=False),
        scratch_types=[
            pltpu.VMEM((E,), jnp.float32),
            pltpu.VMEM((NBLK * BLKS,), jnp.float32),
            pltpu.VMEM((NBLK * BLKS,), jnp.float32),
            pltpu.VMEM((NBLK * 64,), jnp.int32),
            pltpu.VMEM((C // 8, 2, 8, 128), jnp.float32),
            pltpu.VMEM((C // 8, 2, 8, 128), jnp.float32),
            pltpu.VMEM((C * 8 + 8,), jnp.float32),
            pltpu.VMEM((C * 8 + 8,), jnp.int32),
            pltpu.SemaphoreType.DMA,
            pltpu.SemaphoreType.DMA,
            pltpu.SemaphoreType.DMA,
        ],
    )
    xt = logits.reshape(4096, 8, 2, 128).transpose(0, 2, 1, 3)
    vals, idxs = f(xt, e_score_correction_bias)
    return vals.reshape(T, TOP_K), idxs.reshape(T, TOP_K)
